# baseline (speedup 1.0000x reference)
import functools

import jax
import jax.numpy as jnp
from jax import lax
from jax.experimental import pallas as pl
from jax.experimental.pallas import tpu as pltpu, tpu_sc as plsc

MAX_SUBJECTS = 100000
EMBED_DIM = 64
BATCH = 16384

_info = plsc.get_sparse_core_info()
_NC, _NS = _info.num_cores, _info.num_subcores
_NW = _NC * _NS
_B_PER_W = BATCH // _NW

_NCHUNK = 4
_CH = _B_PER_W // _NCHUNK

_mesh = plsc.VectorSubcoreMesh(core_axis_name="c", subcore_axis_name="s")


@functools.partial(
    pl.kernel,
    mesh=_mesh,
    compiler_params=pltpu.CompilerParams(use_tc_tiling_on_sc=False),
    out_type=jax.ShapeDtypeStruct((BATCH, EMBED_DIM), jnp.float32),
    scratch_types=[
        pltpu.VMEM((_NCHUNK, _CH), jnp.int32),
        pltpu.VMEM((_CH, EMBED_DIM), jnp.float32),
        pltpu.VMEM((_CH, EMBED_DIM), jnp.float32),
        pltpu.SemaphoreType.DMA,
        pltpu.SemaphoreType.DMA,
        pltpu.SemaphoreType.DMA,
        pltpu.SemaphoreType.DMA,
    ],
)
def _gather_kernel(idx_hbm, table_hbm, out_hbm, idx_v, buf0, buf1,
                   gsem0, gsem1, wsem0, wsem1):
    wid = lax.axis_index("s") * _NC + lax.axis_index("c")
    base = wid * _B_PER_W
    for c in range(_NCHUNK):
        pltpu.sync_copy(idx_hbm.at[pl.ds(base + c * _CH, _CH)], idx_v.at[c])

    bufs = (buf0, buf1)
    gsems = (gsem0, gsem1)
    wsems = (wsem0, wsem1)
    gcopies = [None] * _NCHUNK
    wcopies = [None, None]

    gcopies[0] = pltpu.async_copy(table_hbm.at[idx_v.at[0]], buf0, gsem0)
    for c in range(_NCHUNK):
        nb = (c + 1) % 2
        if c + 1 < _NCHUNK:
            if wcopies[nb] is not None:
                wcopies[nb].wait()
                wcopies[nb] = None
            gcopies[c + 1] = pltpu.async_copy(
                table_hbm.at[idx_v.at[c + 1]], bufs[nb], gsems[nb])
        gcopies[c].wait()
        wcopies[c % 2] = pltpu.async_copy(
            bufs[c % 2], out_hbm.at[pl.ds(base + c * _CH, _CH)], wsems[c % 2])
    for w in wcopies:
        if w is not None:
            w.wait()


def kernel(subject_ids, embed_weight):
    return _gather_kernel(subject_ids.astype(jnp.int32), embed_weight)


# speedup vs baseline: 1.5252x; 1.5252x over previous
import functools

import jax
import jax.numpy as jnp
from jax import lax
from jax.experimental import pallas as pl
from jax.experimental.pallas import tpu as pltpu, tpu_sc as plsc

MAX_SUBJECTS = 100000
EMBED_DIM = 64
BATCH = 16384

_info = plsc.get_sparse_core_info()
_NC, _NS = _info.num_cores, _info.num_subcores
_NW = _NC * _NS
_B_PER_W = BATCH // _NW

_mesh = plsc.VectorSubcoreMesh(core_axis_name="c", subcore_axis_name="s")


@functools.partial(
    pl.kernel,
    mesh=_mesh,
    out_type=jax.ShapeDtypeStruct((BATCH, EMBED_DIM), jnp.float32),
    scratch_types=[
        pltpu.VMEM((_B_PER_W,), jnp.int32),
        pltpu.VMEM((_B_PER_W, EMBED_DIM), jnp.float32),
        pltpu.SemaphoreType.DMA,
    ],
)
def _gather_kernel(idx_hbm, table_hbm, out_hbm, idx_vm, rows_v, gsem):
    wid = lax.axis_index("s") * _NC + lax.axis_index("c")
    base = wid * _B_PER_W
    pltpu.sync_copy(idx_hbm.at[pl.ds(base, _B_PER_W)], idx_vm)

    def body(g, carry):
        v = idx_vm[pl.ds(g * 16, 16)]
        for j in range(16):
            pltpu.async_copy(
                table_hbm.at[pl.ds(v[j], 1)],
                rows_v.at[pl.ds(g * 16 + j, 1)], gsem)
        return carry

    lax.fori_loop(0, _B_PER_W // 16, body, 0)
    pltpu.make_async_copy(
        table_hbm.at[pl.ds(0, _B_PER_W)], rows_v, gsem).wait()
    pltpu.sync_copy(rows_v, out_hbm.at[pl.ds(base, _B_PER_W)])


def kernel(subject_ids, embed_weight):
    return _gather_kernel(subject_ids.astype(jnp.int32), embed_weight)
